# baseline (speedup 1.0000x reference)
import functools

import jax
import jax.numpy as jnp
from jax import lax
from jax.experimental import pallas as pl
from jax.experimental.pallas import tpu as pltpu
from jax.experimental.pallas import tpu_sc as plsc

MAX_RATING = 5.0
MIN_RATING = 1.0

B = 16384
D = 128
NH = 128

_NC = 2
_NS = 16
_NW = _NC * _NS
_CHUNK = 128
_NBUFMAX = 7

_SEG = 2


@functools.cache
def _make_gather(nrows):
    bpw = nrows // _NW
    nchunk = bpw // _CHUNK
    ngath = 2 * nchunk
    nbuf = min(_NBUFMAX, ngath)
    ipw = nchunk
    mesh = plsc.VectorSubcoreMesh(core_axis_name="c", subcore_axis_name="s")

    @functools.partial(
        pl.kernel,
        mesh=mesh,
        out_type=[
            jax.ShapeDtypeStruct((nrows, D), jnp.float32),
            jax.ShapeDtypeStruct((nrows, D), jnp.float32),
        ],
        scratch_types=(
            [pltpu.VMEM((ipw, _CHUNK), jnp.int32)] * 2
            + [pltpu.VMEM((_CHUNK, D), jnp.float32)] * nbuf
            + [pltpu.SemaphoreType.DMA] * (2 * ngath)
        ),
    )
    def gather2(u_tab, m_tab, users2d, movies2d, u_out, m_out, *scratch):
        iu, im = scratch[:2]
        bufs = scratch[2:2 + nbuf]
        gsems = scratch[2 + nbuf:2 + nbuf + ngath]
        wsems = scratch[2 + nbuf + ngath:]
        wid = lax.axis_index("s") * _NC + lax.axis_index("c")
        base = wid * bpw
        pltpu.sync_copy(users2d.at[pl.ds(wid * ipw, ipw)], iu)
        pltpu.sync_copy(movies2d.at[pl.ds(wid * ipw, ipw)], im)
        specs = [(u_tab, iu, u_out, j) for j in range(nchunk)] + \
                [(m_tab, im, m_out, j) for j in range(nchunk)]
        gathers = [None] * ngath
        writes = [None] * ngath
        for j in range(nbuf):
            tab, idx, _, c = specs[j]
            gathers[j] = pltpu.async_copy(tab.at[idx.at[c]], bufs[j], gsems[j])
        for j in range(ngath):
            if j >= nbuf:
                writes[j - nbuf].wait()
                tab, idx, _, c = specs[j]
                gathers[j] = pltpu.async_copy(tab.at[idx.at[c]],
                                              bufs[j % nbuf], gsems[j])
            gathers[j].wait()
            _, _, out_hbm, c = specs[j]
            writes[j] = pltpu.async_copy(
                bufs[j % nbuf],
                out_hbm.at[pl.ds(base + c * _CHUNK, _CHUNK)], wsems[j])
        for j in range(ngath - nbuf, ngath):
            writes[j].wait()

    return gather2


_TILE = 4096

_DNUM = (((1,), (1,)), ((), ()))


def _mlp_body(u2_ref, m2_ref, w1_ref, b1_ref, w2_ref, b2_ref, out_ref):
    w1 = w1_ref[...]
    h = (lax.dot_general(u2_ref[...], w1[:, :D], _DNUM,
                         preferred_element_type=jnp.float32)
         + lax.dot_general(m2_ref[...], w1[:, D:], _DNUM,
                           preferred_element_type=jnp.float32)
         + b1_ref[...])
    h = jnp.maximum(h, 0.0)
    z = jnp.sum(h * w2_ref[...], axis=1) + b2_ref[0, 0]
    r = jax.nn.sigmoid(z) * (MAX_RATING - MIN_RATING) + MIN_RATING
    out_ref[...] = r.reshape(_TILE // 128, 128)


def _mlp(u2, m2, w1, b1, w2, b2):
    nrows = u2.shape[0]
    grid = (nrows // _TILE,)
    return pl.pallas_call(
        _mlp_body,
        grid=grid,
        in_specs=[
            pl.BlockSpec((_TILE, D), lambda i: (i, 0)),
            pl.BlockSpec((_TILE, D), lambda i: (i, 0)),
            pl.BlockSpec((NH, 2 * D), lambda i: (0, 0)),
            pl.BlockSpec((1, NH), lambda i: (0, 0)),
            pl.BlockSpec((1, NH), lambda i: (0, 0)),
            pl.BlockSpec((1, 1), lambda i: (0, 0)),
        ],
        out_specs=pl.BlockSpec((_TILE // 128, 128), lambda i: (i, 0)),
        out_shape=jax.ShapeDtypeStruct((nrows // 128, 128), jnp.float32),
    )(u2, m2, w1, b1, w2, b2)


def kernel(users, movies, u_weight, m_weight, lin1_w, lin1_b, lin2_w, lin2_b):
    b1 = lin1_b.reshape(1, NH)
    w2 = lin2_w.reshape(1, NH)
    b2 = lin2_b.reshape(1, 1)
    users2d = users.reshape(B // _CHUNK, _CHUNK)
    movies2d = movies.reshape(B // _CHUNK, _CHUNK)
    nrows = B // _SEG
    irows = nrows // _CHUNK
    gather = _make_gather(nrows)
    halves = [
        gather(u_weight, m_weight,
               users2d[k * irows:(k + 1) * irows],
               movies2d[k * irows:(k + 1) * irows])
        for k in range(_SEG)
    ]
    outs = [_mlp(u2, m2, lin1_w, b1, w2, b2) for u2, m2 in halves]
    return jnp.concatenate(outs, axis=0).reshape(B, 1)


# speedup vs baseline: 1.1127x; 1.1127x over previous
import functools

import jax
import jax.numpy as jnp
from jax import lax
from jax.experimental import pallas as pl
from jax.experimental.pallas import tpu as pltpu
from jax.experimental.pallas import tpu_sc as plsc

MAX_RATING = 5.0
MIN_RATING = 1.0

B = 16384
D = 128
NH = 128

_NC = 2
_NS = 16
_NW = _NC * _NS
_CHUNK = 128
_NBUFMAX = 7

_SEG = 1


@functools.cache
def _make_gather(nrows):
    bpw = nrows // _NW
    nchunk = bpw // _CHUNK
    ngath = 2 * nchunk
    nbuf = min(_NBUFMAX, ngath)
    ipw = nchunk
    mesh = plsc.VectorSubcoreMesh(core_axis_name="c", subcore_axis_name="s")

    @functools.partial(
        pl.kernel,
        mesh=mesh,
        out_type=[
            jax.ShapeDtypeStruct((nrows, D), jnp.float32),
            jax.ShapeDtypeStruct((nrows, D), jnp.float32),
        ],
        scratch_types=(
            [pltpu.VMEM((ipw, _CHUNK), jnp.int32)] * 2
            + [pltpu.VMEM((_CHUNK, D), jnp.float32)] * nbuf
            + [pltpu.SemaphoreType.DMA] * (2 * ngath)
        ),
    )
    def gather2(u_tab, m_tab, users2d, movies2d, u_out, m_out, *scratch):
        iu, im = scratch[:2]
        bufs = scratch[2:2 + nbuf]
        gsems = scratch[2 + nbuf:2 + nbuf + ngath]
        wsems = scratch[2 + nbuf + ngath:]
        wid = lax.axis_index("s") * _NC + lax.axis_index("c")
        base = wid * bpw
        pltpu.sync_copy(users2d.at[pl.ds(wid * ipw, ipw)], iu)
        pltpu.sync_copy(movies2d.at[pl.ds(wid * ipw, ipw)], im)
        specs = [(u_tab, iu, u_out, j) for j in range(nchunk)] + \
                [(m_tab, im, m_out, j) for j in range(nchunk)]
        gathers = [None] * ngath
        writes = [None] * ngath
        for j in range(nbuf):
            tab, idx, _, c = specs[j]
            gathers[j] = pltpu.async_copy(tab.at[idx.at[c]], bufs[j], gsems[j])
        for j in range(ngath):
            if j >= nbuf:
                writes[j - nbuf].wait()
                tab, idx, _, c = specs[j]
                gathers[j] = pltpu.async_copy(tab.at[idx.at[c]],
                                              bufs[j % nbuf], gsems[j])
            gathers[j].wait()
            _, _, out_hbm, c = specs[j]
            writes[j] = pltpu.async_copy(
                bufs[j % nbuf],
                out_hbm.at[pl.ds(base + c * _CHUNK, _CHUNK)], wsems[j])
        for j in range(ngath - nbuf, ngath):
            writes[j].wait()

    return gather2


_TILE = 8192

_DNUM = (((1,), (1,)), ((), ()))


def _mlp_body(u2_ref, m2_ref, w1_ref, b1_ref, w2_ref, b2_ref, out_ref):
    w1 = w1_ref[...]
    h = (lax.dot_general(u2_ref[...], w1[:, :D], _DNUM,
                         preferred_element_type=jnp.float32)
         + lax.dot_general(m2_ref[...], w1[:, D:], _DNUM,
                           preferred_element_type=jnp.float32)
         + b1_ref[...])
    h = jnp.maximum(h, 0.0)
    z = jnp.sum(h * w2_ref[...], axis=1) + b2_ref[0, 0]
    r = jax.nn.sigmoid(z) * (MAX_RATING - MIN_RATING) + MIN_RATING
    out_ref[...] = r.reshape(_TILE // 128, 128)


def _mlp(u2, m2, w1, b1, w2, b2):
    nrows = u2.shape[0]
    grid = (nrows // _TILE,)
    return pl.pallas_call(
        _mlp_body,
        grid=grid,
        in_specs=[
            pl.BlockSpec((_TILE, D), lambda i: (i, 0)),
            pl.BlockSpec((_TILE, D), lambda i: (i, 0)),
            pl.BlockSpec((NH, 2 * D), lambda i: (0, 0)),
            pl.BlockSpec((1, NH), lambda i: (0, 0)),
            pl.BlockSpec((1, NH), lambda i: (0, 0)),
            pl.BlockSpec((1, 1), lambda i: (0, 0)),
        ],
        out_specs=pl.BlockSpec((_TILE // 128, 128), lambda i: (i, 0)),
        out_shape=jax.ShapeDtypeStruct((nrows // 128, 128), jnp.float32),
    )(u2, m2, w1, b1, w2, b2)


def kernel(users, movies, u_weight, m_weight, lin1_w, lin1_b, lin2_w, lin2_b):
    b1 = lin1_b.reshape(1, NH)
    w2 = lin2_w.reshape(1, NH)
    b2 = lin2_b.reshape(1, 1)
    users2d = users.reshape(B // _CHUNK, _CHUNK)
    movies2d = movies.reshape(B // _CHUNK, _CHUNK)
    nrows = B // _SEG
    irows = nrows // _CHUNK
    gather = _make_gather(nrows)
    halves = [
        gather(u_weight, m_weight,
               users2d[k * irows:(k + 1) * irows],
               movies2d[k * irows:(k + 1) * irows])
        for k in range(_SEG)
    ]
    outs = [_mlp(u2, m2, lin1_w, b1, w2, b2) for u2, m2 in halves]
    return jnp.concatenate(outs, axis=0).reshape(B, 1)


# speedup vs baseline: 1.1277x; 1.0135x over previous
import functools

import jax
import jax.numpy as jnp
from jax import lax
from jax.experimental import pallas as pl
from jax.experimental.pallas import tpu as pltpu
from jax.experimental.pallas import tpu_sc as plsc

MAX_RATING = 5.0
MIN_RATING = 1.0

B = 16384
D = 128
NH = 128

_NC = 2
_NS = 16
_NW = _NC * _NS
_CHUNK = 128
_NBUFMAX = 7

_SEG = 1


@functools.cache
def _make_gather(nrows):
    bpw = nrows // _NW
    nchunk = bpw // _CHUNK
    ngath = 2 * nchunk
    nbuf = min(_NBUFMAX, ngath)
    ipw = nchunk
    mesh = plsc.VectorSubcoreMesh(core_axis_name="c", subcore_axis_name="s")

    @functools.partial(
        pl.kernel,
        mesh=mesh,
        out_type=[
            jax.ShapeDtypeStruct((nrows, D), jnp.float32),
            jax.ShapeDtypeStruct((nrows, D), jnp.float32),
        ],
        scratch_types=(
            [pltpu.VMEM((ipw, _CHUNK), jnp.int32)] * 2
            + [pltpu.VMEM((_CHUNK, D), jnp.float32)] * nbuf
            + [pltpu.SemaphoreType.DMA] * (2 * ngath)
        ),
    )
    def gather2(u_tab, m_tab, users2d, movies2d, u_out, m_out, *scratch):
        iu, im = scratch[:2]
        bufs = scratch[2:2 + nbuf]
        gsems = scratch[2 + nbuf:2 + nbuf + ngath]
        wsems = scratch[2 + nbuf + ngath:]
        wid = lax.axis_index("s") * _NC + lax.axis_index("c")
        base = wid * bpw
        icp_u = pltpu.async_copy(users2d.at[pl.ds(wid * ipw, ipw)], iu,
                                 gsems[0])
        icp_m = pltpu.async_copy(movies2d.at[pl.ds(wid * ipw, ipw)], im,
                                 gsems[1])
        icp_u.wait()
        icp_m.wait()
        specs = [spec
                 for j in range(nchunk)
                 for spec in ((u_tab, iu, u_out, j), (m_tab, im, m_out, j))]
        gathers = [None] * ngath
        writes = [None] * ngath
        for j in range(nbuf):
            tab, idx, _, c = specs[j]
            gathers[j] = pltpu.async_copy(tab.at[idx.at[c]], bufs[j], gsems[j])
        for j in range(ngath):
            if j >= nbuf:
                writes[j - nbuf].wait()
                tab, idx, _, c = specs[j]
                gathers[j] = pltpu.async_copy(tab.at[idx.at[c]],
                                              bufs[j % nbuf], gsems[j])
            gathers[j].wait()
            _, _, out_hbm, c = specs[j]
            writes[j] = pltpu.async_copy(
                bufs[j % nbuf],
                out_hbm.at[pl.ds(base + c * _CHUNK, _CHUNK)], wsems[j])
        for j in range(ngath - nbuf, ngath):
            writes[j].wait()

    return gather2


_TILE = 8192

_DNUM = (((1,), (1,)), ((), ()))


def _mlp_body(u2_ref, m2_ref, w1_ref, b1_ref, w2_ref, b2_ref, out_ref):
    w1 = w1_ref[...]
    h = (lax.dot_general(u2_ref[...], w1[:, :D], _DNUM,
                         preferred_element_type=jnp.float32)
         + lax.dot_general(m2_ref[...], w1[:, D:], _DNUM,
                           preferred_element_type=jnp.float32)
         + b1_ref[...])
    h = jnp.maximum(h, 0.0)
    z = jnp.sum(h * w2_ref[...], axis=1) + b2_ref[0, 0]
    r = jax.nn.sigmoid(z) * (MAX_RATING - MIN_RATING) + MIN_RATING
    out_ref[...] = r.reshape(_TILE // 128, 128)


def _mlp(u2, m2, w1, b1, w2, b2):
    nrows = u2.shape[0]
    grid = (nrows // _TILE,)
    return pl.pallas_call(
        _mlp_body,
        grid=grid,
        in_specs=[
            pl.BlockSpec((_TILE, D), lambda i: (i, 0)),
            pl.BlockSpec((_TILE, D), lambda i: (i, 0)),
            pl.BlockSpec((NH, 2 * D), lambda i: (0, 0)),
            pl.BlockSpec((1, NH), lambda i: (0, 0)),
            pl.BlockSpec((1, NH), lambda i: (0, 0)),
            pl.BlockSpec((1, 1), lambda i: (0, 0)),
        ],
        out_specs=pl.BlockSpec((_TILE // 128, 128), lambda i: (i, 0)),
        out_shape=jax.ShapeDtypeStruct((nrows // 128, 128), jnp.float32),
    )(u2, m2, w1, b1, w2, b2)


def kernel(users, movies, u_weight, m_weight, lin1_w, lin1_b, lin2_w, lin2_b):
    b1 = lin1_b.reshape(1, NH)
    w2 = lin2_w.reshape(1, NH)
    b2 = lin2_b.reshape(1, 1)
    users2d = users.reshape(B // _CHUNK, _CHUNK)
    movies2d = movies.reshape(B // _CHUNK, _CHUNK)
    nrows = B // _SEG
    irows = nrows // _CHUNK
    gather = _make_gather(nrows)
    halves = [
        gather(u_weight, m_weight,
               users2d[k * irows:(k + 1) * irows],
               movies2d[k * irows:(k + 1) * irows])
        for k in range(_SEG)
    ]
    outs = [_mlp(u2, m2, lin1_w, b1, w2, b2) for u2, m2 in halves]
    return jnp.concatenate(outs, axis=0).reshape(B, 1)
